# baseline (speedup 1.0000x reference)
import functools

import jax
import jax.numpy as jnp
from jax import lax
from jax.experimental import pallas as pl
from jax.experimental.pallas import tpu as pltpu
from jax.experimental.pallas import tpu_sc as plsc

D_MODEL = 64
SCALE = 8.0
VOCAB = 1000000
NC, NS = 2, 16
NW = NC * NS

B0, B1 = 4096, 200
B = B0 * B1
ROWS_PW = B // NW
CHUNK = 128
NCHUNK = ROWS_PW // CHUNK

VCHUNK = 4096


def _stage_body(a_ref, o_ref):
    b = a_ref[...].T * SCALE
    o_ref[...] = jnp.concatenate(
        [b[: VCHUNK // 2], b[VCHUNK // 2:]], axis=1)


@jax.jit
def _stage(tt):
    return pl.pallas_call(
        _stage_body,
        grid=(pl.cdiv(VOCAB, VCHUNK),),
        in_specs=[pl.BlockSpec((D_MODEL, VCHUNK), lambda c: (0, c))],
        out_specs=pl.BlockSpec((VCHUNK // 2, 128), lambda c: (c, 0)),
        out_shape=jax.ShapeDtypeStruct(
            (pl.cdiv(VOCAB, VCHUNK) * (VCHUNK // 2), 128), jnp.float32),
    )(tt)


@functools.partial(
    pl.kernel,
    out_type=jax.ShapeDtypeStruct((B, 128), jnp.float32),
    mesh=plsc.VectorSubcoreMesh(
        core_axis_name="c", subcore_axis_name="s",
        num_cores=NC, num_subcores=NS),
    scratch_types=[
        pltpu.VMEM((3, CHUNK), jnp.int32),
        pltpu.VMEM((3, CHUNK, 128), jnp.float32),
        pltpu.VMEM((3, CHUNK, 128), jnp.float32),
        pltpu.VMEM((3, CHUNK), jnp.int32),
        pltpu.SemaphoreType.DMA((3,)),
        pltpu.SemaphoreType.DMA((3,)),
        pltpu.SemaphoreType.DMA((3,)),
        pltpu.SemaphoreType.DMA((3,)),
    ],
    compiler_params=pltpu.CompilerParams(
        use_tc_tiling_on_sc=True, needs_layout_passes=False),
)
def _gather(xflat, xrow, s, out, idxv, dst, packed, idx2v, isem, gsem, osem, rsem):
    w = lax.axis_index("s") * NC + lax.axis_index("c")
    base = w * ROWS_PW

    def idx_copy(t, b):
        return pltpu.make_async_copy(
            xflat.at[pl.ds(base + t * CHUNK, CHUNK)], idxv.at[b], isem.at[b])

    def row_copy(t, b):
        return pltpu.make_async_copy(
            xrow.at[pl.ds(base + t * CHUNK, CHUNK)], idx2v.at[b], rsem.at[b])

    def gather_copy(b):
        return pltpu.make_async_copy(s.at[idx2v.at[b]], dst.at[b], gsem.at[b])

    def out_copy(t, b):
        return pltpu.make_async_copy(
            packed.at[b], out.at[pl.ds(base + t * CHUNK, CHUNK)], osem.at[b])

    def repack(b):
        @plsc.parallel_loop(0, CHUNK, unroll=4, carry=jnp.int32(0))
        def _(j, c):
            seg = idxv[b, pl.ds(lax.div(j, 16) * 16, 16)]
            lane = jnp.full((16,), lax.rem(j, 16), jnp.int32)
            odd = (jax.lax.shift_right_logical(seg[lane], 11) & 1) == 1
            for k in range(4):
                lo = dst[b, j, pl.ds(k * 16, 16)]
                hi = dst[b, j, pl.ds(D_MODEL + k * 16, 16)]
                packed[b, j, pl.ds(k * 16, 16)] = jnp.where(odd, hi, lo)
            return c

    idx_copy(0, 0).start()
    row_copy(0, 0).start()
    idx_copy(1, 1).start()
    row_copy(1, 1).start()
    row_copy(0, 0).wait()
    gather_copy(0).start()
    row_copy(1, 1).wait()
    gather_copy(1).start()
    idx_copy(2, 2).start()
    row_copy(2, 2).start()

    def body(t, carry):
        b = lax.rem(t, 3)
        b2 = lax.rem(t + 2, 3)

        gather_copy(b).wait()

        @pl.when(t + 2 < NCHUNK)
        def _():
            row_copy(t + 2, b2).wait()
            gather_copy(b2).start()

        @pl.when(t + 3 < NCHUNK)
        def _():
            idx_copy(t + 3, b).start()
            row_copy(t + 3, b).start()

        @pl.when(t >= 3)
        def _():
            out_copy(t - 3, b).wait()

        idx_copy(t, b).wait()
        repack(b)
        out_copy(t, b).start()
        return carry

    lax.fori_loop(0, NCHUNK, body, 0)
    out_copy(NCHUNK - 3, (NCHUNK - 3) % 3).wait()
    out_copy(NCHUNK - 2, (NCHUNK - 2) % 3).wait()
    out_copy(NCHUNK - 1, (NCHUNK - 1) % 3).wait()


def kernel(x, lut_weight):
    tt = lut_weight.T
    staged = _stage(tt)
    xflat = x.astype(jnp.int32).reshape(B)
    xrow = ((xflat >> 12) << 11) + (xflat & (VCHUNK // 2 - 1))
    out = _gather(xflat, xrow, staged)
    return out.reshape(B0, B1, 128)[:, :, :D_MODEL]


# speedup vs baseline: 1.0003x; 1.0003x over previous
import functools

import jax
import jax.numpy as jnp
from jax import lax
from jax.experimental import pallas as pl
from jax.experimental.pallas import tpu as pltpu
from jax.experimental.pallas import tpu_sc as plsc

D_MODEL = 64
SCALE = 8.0
VOCAB = 1000000
NC, NS = 2, 16
NW = NC * NS

B0, B1 = 4096, 200
B = B0 * B1
ROWS_PW = B // NW
CHUNK = 128
NCHUNK = ROWS_PW // CHUNK

VCHUNK = 4096


def _stage_body(a_ref, o_ref):
    b = a_ref[...].T * SCALE
    o_ref[...] = jnp.concatenate(
        [b[: VCHUNK // 2], b[VCHUNK // 2:]], axis=1)


@jax.jit
def _stage(tt):
    return pl.pallas_call(
        _stage_body,
        grid=(pl.cdiv(VOCAB, VCHUNK),),
        in_specs=[pl.BlockSpec((D_MODEL, VCHUNK), lambda c: (0, c))],
        out_specs=pl.BlockSpec((VCHUNK // 2, 128), lambda c: (c, 0)),
        out_shape=jax.ShapeDtypeStruct(
            (pl.cdiv(VOCAB, VCHUNK) * (VCHUNK // 2), 128), jnp.float32),
    )(tt)


@functools.partial(
    pl.kernel,
    out_type=jax.ShapeDtypeStruct((B, 128), jnp.float32),
    mesh=plsc.VectorSubcoreMesh(
        core_axis_name="c", subcore_axis_name="s",
        num_cores=NC, num_subcores=NS),
    scratch_types=[
        pltpu.VMEM((3, CHUNK), jnp.int32),
        pltpu.VMEM((3, CHUNK, 128), jnp.float32),
        pltpu.VMEM((3, CHUNK, 128), jnp.float32),
        pltpu.VMEM((3, CHUNK), jnp.int32),
        pltpu.SemaphoreType.DMA((3,)),
        pltpu.SemaphoreType.DMA((3,)),
        pltpu.SemaphoreType.DMA((3,)),
        pltpu.SemaphoreType.DMA((3,)),
    ],
    compiler_params=pltpu.CompilerParams(
        use_tc_tiling_on_sc=True, needs_layout_passes=False),
)
def _gather(xflat, xrow, s, out, idxv, dst, packed, idx2v, isem, gsem, osem, rsem):
    w = lax.axis_index("s") * NC + lax.axis_index("c")
    base = w * ROWS_PW

    def idx_copy(t, b):
        return pltpu.make_async_copy(
            xflat.at[pl.ds(base + t * CHUNK, CHUNK)], idxv.at[b], isem.at[b])

    def row_copy(t, b):
        return pltpu.make_async_copy(
            xrow.at[pl.ds(base + t * CHUNK, CHUNK)], idx2v.at[b], rsem.at[b])

    def gather_copy(b):
        return pltpu.make_async_copy(s.at[idx2v.at[b]], dst.at[b], gsem.at[b])

    def out_copy(t, b):
        return pltpu.make_async_copy(
            packed.at[b], out.at[pl.ds(base + t * CHUNK, CHUNK)], osem.at[b])

    def repack(b):
        lanes = lax.iota(jnp.int32, 16)

        @plsc.parallel_loop(0, CHUNK, unroll=4, carry=jnp.int32(0))
        def _(j, c):
            seg = idxv[b, pl.ds(lax.div(j, 16) * 16, 16)]
            bits = jax.lax.shift_right_logical(seg, 11) & 1
            sel = jnp.sum(jnp.where(lanes == lax.rem(j, 16), bits, 0))
            odd = sel == 1
            for k in range(4):
                lo = dst[b, j, pl.ds(k * 16, 16)]
                hi = dst[b, j, pl.ds(D_MODEL + k * 16, 16)]
                packed[b, j, pl.ds(k * 16, 16)] = jnp.where(odd, hi, lo)
            return c

    idx_copy(0, 0).start()
    row_copy(0, 0).start()
    idx_copy(1, 1).start()
    row_copy(1, 1).start()
    row_copy(0, 0).wait()
    gather_copy(0).start()
    row_copy(1, 1).wait()
    gather_copy(1).start()
    idx_copy(2, 2).start()
    row_copy(2, 2).start()

    def body(t, carry):
        b = lax.rem(t, 3)
        b2 = lax.rem(t + 2, 3)

        gather_copy(b).wait()
        idx_copy(t, b).wait()

        @pl.when(t + 2 < NCHUNK)
        def _():
            row_copy(t + 2, b2).wait()
            gather_copy(b2).start()

        @pl.when(t >= 3)
        def _():
            out_copy(t - 3, b).wait()

        repack(b)
        out_copy(t, b).start()

        @pl.when(t + 3 < NCHUNK)
        def _():
            idx_copy(t + 3, b).start()
            row_copy(t + 3, b).start()

        return carry

    lax.fori_loop(0, NCHUNK, body, 0)
    out_copy(NCHUNK - 3, (NCHUNK - 3) % 3).wait()
    out_copy(NCHUNK - 2, (NCHUNK - 2) % 3).wait()
    out_copy(NCHUNK - 1, (NCHUNK - 1) % 3).wait()


def kernel(x, lut_weight):
    tt = lut_weight.T
    staged = _stage(tt)
    xflat = x.astype(jnp.int32).reshape(B)
    xrow = ((xflat >> 12) << 11) + (xflat & (VCHUNK // 2 - 1))
    out = _gather(xflat, xrow, staged)
    return out.reshape(B0, B1, 128)[:, :, :D_MODEL]
